# baseline (speedup 1.0000x reference)
import jax
import jax.numpy as jnp
from jax import lax
from jax.experimental import pallas as pl
from jax.experimental.pallas import tpu as pltpu
from jax.experimental.pallas import tpu_sc as plsc

_VOCAB = 1000000
_EMBED_DIM = 32
_BATCH = 4096
_HIST = 200

_NW = 32
_NTILE_FULL = _VOCAB // 128
_TAIL0 = _NTILE_FULL * 128
_NTAIL = _VOCAB - _TAIL0
_RM_ROWS = _VOCAB // 4

_NHG = _HIST // 8
_NBT = _BATCH // 128
_NGRP = _NHG * _NBT
_GPW = _NGRP // _NW


def _detile_body(tT_hbm, tail_hbm, rm_hbm, in_v, out_v, tail_v, sem):
    wid = lax.axis_index("s") * 2 + lax.axis_index("c")

    iota = lax.iota(jnp.int32, 16)

    @pl.when(wid == _NW - 1)
    def _():
        pltpu.sync_copy(tail_hbm, tail_v)
        pltpu.sync_copy(tail_v, rm_hbm.at[pl.ds(_RM_ROWS - _NTAIL // 4, _NTAIL // 4), :])

    def step(k, carry):
        c = k * _NW + wid

        @pl.when(c < _NTILE_FULL)
        def _():
            pltpu.async_copy(
                tT_hbm.at[:, pl.ds(c * 128, 128)], in_v, sem).wait()
            for r in range(32):
                for p in range(8):
                    q = p // 2
                    e_idx = iota + 16 * (p % 2)
                    c_idx = jnp.full((16,), 4 * r + q, jnp.int32)
                    vals = plsc.load_gather(in_v, [e_idx, c_idx])
                    out_v[r, pl.ds(16 * p, 16)] = vals
            pltpu.async_copy(out_v, rm_hbm.at[pl.ds(c * 32, 32), :], sem).wait()
        return carry

    lax.fori_loop(0, (_NTILE_FULL + _NW - 1) // _NW, step, 0)


def _gather_body(rm_hbm, x3_hbm, out_hbm, idx_v, rows_v, ob_v, sem_i, sem_g, sem_o):
    wid = lax.axis_index("s") * 2 + lax.axis_index("c")

    iota = lax.iota(jnp.int32, 16)

    def step(g, carry):
        grp = wid * _GPW + g
        hg = grp // _NBT
        t = lax.rem(grp, _NBT)
        pltpu.async_copy(x3_hbm.at[pl.ds(hg * 8, 8), t, :], idx_v, sem_i).wait()
        for s in range(8):
            pltpu.async_copy(rm_hbm.at[idx_v.at[s]], rows_v, sem_g).wait()
            for r in range(4):
                for s2 in range(8):
                    for p in range(8):
                        r_idx = iota + 16 * p
                        c_idx = jnp.full((16,), 8 * r + s2, jnp.int32)
                        vals = plsc.load_gather(rows_v, [r_idx, c_idx])
                        ob_v[r, s2, pl.ds(16 * p, 16)] = vals
            pltpu.async_copy(
                ob_v, out_hbm.at[hg * 8 + s, :, t], sem_o).wait()
        return carry

    lax.fori_loop(0, _GPW, step, 0)


@jax.jit
def _embedding(x, table):
    mesh = plsc.VectorSubcoreMesh(core_axis_name="c", subcore_axis_name="s")

    detile = pl.kernel(
        _detile_body,
        out_type=jax.ShapeDtypeStruct((_RM_ROWS, 128), jnp.float32),
        mesh=mesh,
        scratch_types=[
            pltpu.VMEM((32, 128), jnp.float32),
            pltpu.VMEM((32, 128), jnp.float32),
            pltpu.VMEM((_NTAIL // 4, 128), jnp.float32),
            pltpu.SemaphoreType.DMA,
        ],
        compiler_params=pltpu.CompilerParams(
            use_tc_tiling_on_sc=True, needs_layout_passes=False),
    )

    gather = pl.kernel(
        _gather_body,
        out_type=jax.ShapeDtypeStruct((_HIST, 4, _NBT, 8, 128), jnp.float32),
        mesh=mesh,
        scratch_types=[
            pltpu.VMEM((8, 128), jnp.int32),
            pltpu.VMEM((128, _EMBED_DIM), jnp.float32),
            pltpu.VMEM((4, 8, 128), jnp.float32),
            pltpu.SemaphoreType.DMA,
            pltpu.SemaphoreType.DMA,
            pltpu.SemaphoreType.DMA,
        ],
        compiler_params=pltpu.CompilerParams(
            use_tc_tiling_on_sc=False, needs_layout_passes=False),
    )

    tT = table.T
    tail = table[_TAIL0:, :].reshape(_NTAIL // 4, 128)
    rm = detile(tT, tail)
    rm_lin = rm.reshape(_VOCAB, _EMBED_DIM)
    x3 = x.T.reshape(_HIST, _NBT, 128)
    out_lin = gather(rm_lin, x3)
    return out_lin.transpose(2, 4, 0, 1, 3).reshape(_BATCH, _HIST, _EMBED_DIM)


def kernel(x, table):
    return _embedding(x, table)


# speedup vs baseline: 1.2538x; 1.2538x over previous
import jax
import jax.numpy as jnp
from jax import lax
from jax.experimental import pallas as pl
from jax.experimental.pallas import tpu as pltpu
from jax.experimental.pallas import tpu_sc as plsc

_VOCAB = 1000000
_EMBED_DIM = 32
_BATCH = 4096
_HIST = 200

_NW = 32
_NTILE_FULL = _VOCAB // 128
_TAIL0 = _NTILE_FULL * 128
_NTAIL = _VOCAB - _TAIL0
_RM_ROWS = _VOCAB // 4
_KMAX = (_NTILE_FULL + _NW - 1) // _NW

_NHG = _HIST // 8
_NBT = _BATCH // 128
_NGRP = _NHG * _NBT
_GPW = _NGRP // _NW


def _detile_body(tT_hbm, tail_hbm, rm_hbm, in_v, out_v, sems_i, sems_o,
                 tail_v):
    wid = lax.axis_index("s") * 2 + lax.axis_index("c")

    iota = lax.iota(jnp.int32, 16)
    e_idx = (iota, iota + 16)

    @pl.when(wid == _NW - 1)
    def _():
        pltpu.sync_copy(tail_hbm, tail_v)
        pltpu.sync_copy(
            tail_v, rm_hbm.at[pl.ds(_RM_ROWS - _NTAIL // 4, _NTAIL // 4), :])

    def col(k):
        return k * _NW + wid

    def issue_in(k, b):
        pltpu.async_copy(
            tT_hbm.at[:, pl.ds(col(k) * 128, 128)], in_v[b], sems_i[b])

    def wait_in(b):
        pltpu.make_async_copy(
            tT_hbm.at[:, pl.ds(0, 128)], in_v[b], sems_i[b]).wait()

    def issue_out(k, b):
        pltpu.async_copy(
            out_v[b], rm_hbm.at[pl.ds(col(k) * 32, 32), :], sems_o[b])

    def wait_out(b):
        pltpu.make_async_copy(
            out_v[b], rm_hbm.at[pl.ds(0, 32), :], sems_o[b]).wait()

    def transpose(b):
        for r in range(32):
            for p in range(8):
                c_idx = jnp.full((16,), 4 * r + p // 2, jnp.int32)
                out_v[b][r, pl.ds(16 * p, 16)] = plsc.load_gather(
                    in_v[b], [e_idx[p % 2], c_idx])

    issue_in(0, 0)

    def step(k2, carry):
        for par in range(2):
            k = k2 * 2 + par

            @pl.when(col(k) < _NTILE_FULL)
            def _():
                wait_in(par)

                @pl.when(col(k + 1) < _NTILE_FULL)
                def _():
                    issue_in(k + 1, 1 - par)

                @pl.when(k >= 2)
                def _():
                    wait_out(par)
                transpose(par)
                issue_out(k, par)
        return carry

    lax.fori_loop(0, _KMAX // 2, step, 0)
    k_last = _KMAX - 1
    p_last = k_last % 2

    @pl.when(col(k_last) < _NTILE_FULL)
    def _():
        wait_in(p_last)
        wait_out(p_last)
        transpose(p_last)
        issue_out(k_last, p_last)

    wait_out(0)
    wait_out(1)


def _gather_body(rm_hbm, x3_hbm, out_hbm, idx_v, rows_v, ob_v,
                 sem_i, sems_g, sems_o):
    wid = lax.axis_index("s") * 2 + lax.axis_index("c")

    iota = lax.iota(jnp.int32, 16)
    r_idx = tuple(iota + 16 * p for p in range(8))

    def issue_idx(g):
        grp = wid * _GPW + g
        pltpu.async_copy(
            x3_hbm.at[pl.ds((grp // _NBT) * 8, 8), lax.rem(grp, _NBT), :],
            idx_v.at[lax.rem(g, 2)], sem_i)

    def wait_idx():
        pltpu.make_async_copy(
            x3_hbm.at[pl.ds(0, 8), 0, :], idx_v.at[0], sem_i).wait()

    def issue_gather(g, s, par):
        pltpu.async_copy(
            rm_hbm.at[idx_v.at[lax.rem(g, 2), s]], rows_v[par], sems_g[par])

    def wait_gather(par):
        pltpu.make_async_copy(
            rm_hbm.at[idx_v.at[0, 0]], rows_v[par], sems_g[par]).wait()

    def issue_out(g, s, par):
        grp = wid * _GPW + g
        pltpu.async_copy(
            ob_v[par],
            out_hbm.at[(grp // _NBT) * 8 + s, :, lax.rem(grp, _NBT)],
            sems_o[par])

    def wait_out(par):
        pltpu.make_async_copy(
            ob_v[par], out_hbm.at[0, :, 0], sems_o[par]).wait()

    def transpose(par):
        for r in range(4):
            for s2 in range(8):
                c_idx = jnp.full((16,), 8 * r + s2, jnp.int32)
                for p in range(8):
                    ob_v[par][r, s2, pl.ds(16 * p, 16)] = \
                        plsc.load_gather(rows_v[par], [r_idx[p], c_idx])

    grp0 = wid * _GPW
    pltpu.sync_copy(
        x3_hbm.at[pl.ds((grp0 // _NBT) * 8, 8), lax.rem(grp0, _NBT), :],
        idx_v.at[0])
    issue_gather(0, 0, 0)
    issue_gather(0, 1, 1)

    def gstep(g, carry):
        @pl.when(g + 1 < _GPW)
        def _():
            issue_idx(g + 1)
        for s in range(8):
            par = s % 2
            wait_gather(par)

            @pl.when(8 * g + s >= 2)
            def _():
                wait_out(par)
            transpose(par)
            issue_out(g, s, par)
            if s == 6:
                @pl.when(g + 1 < _GPW)
                def _():
                    wait_idx()
            if s < 6:
                @pl.when(8 * g + s + 2 < 8 * _GPW)
                def _():
                    issue_gather(g, s + 2, par)
            else:
                @pl.when(g + 1 < _GPW)
                def _():
                    issue_gather(g + 1, s - 6, par)
        return carry

    lax.fori_loop(0, _GPW, gstep, 0)

    wait_out(0)
    wait_out(1)


@jax.jit
def _embedding(x, table):
    mesh = plsc.VectorSubcoreMesh(core_axis_name="c", subcore_axis_name="s")

    detile = pl.kernel(
        _detile_body,
        out_type=jax.ShapeDtypeStruct((_RM_ROWS, 128), jnp.float32),
        mesh=mesh,
        scratch_types=[
            [pltpu.VMEM((32, 128), jnp.float32) for _ in range(2)],
            [pltpu.VMEM((32, 128), jnp.float32) for _ in range(2)],
            [pltpu.SemaphoreType.DMA for _ in range(2)],
            [pltpu.SemaphoreType.DMA for _ in range(2)],
            pltpu.VMEM((_NTAIL // 4, 128), jnp.float32),
        ],
        compiler_params=pltpu.CompilerParams(
            use_tc_tiling_on_sc=True, needs_layout_passes=False),
    )

    gather = pl.kernel(
        _gather_body,
        out_type=jax.ShapeDtypeStruct((_HIST, 4, _NBT, 8, 128), jnp.float32),
        mesh=mesh,
        scratch_types=[
            pltpu.VMEM((2, 8, 128), jnp.int32),
            [pltpu.VMEM((128, _EMBED_DIM), jnp.float32) for _ in range(2)],
            [pltpu.VMEM((4, 8, 128), jnp.float32) for _ in range(2)],
            pltpu.SemaphoreType.DMA,
            [pltpu.SemaphoreType.DMA for _ in range(2)],
            [pltpu.SemaphoreType.DMA for _ in range(2)],
        ],
        compiler_params=pltpu.CompilerParams(
            use_tc_tiling_on_sc=False, needs_layout_passes=False),
    )

    tT = table.T
    tail = table[_TAIL0:, :].reshape(_NTAIL // 4, 128)
    rm = detile(tT, tail)
    rm_lin = rm.reshape(_VOCAB, _EMBED_DIM)
    x3 = x.T.reshape(_HIST, _NBT, 128)
    out_lin = gather(rm_lin, x3)
    return out_lin.transpose(2, 4, 0, 1, 3).reshape(_BATCH, _HIST, _EMBED_DIM)


def kernel(x, table):
    return _embedding(x, table)


# speedup vs baseline: 1.7184x; 1.3705x over previous
import jax
import jax.numpy as jnp
from jax import lax
from jax.experimental import pallas as pl
from jax.experimental.pallas import tpu as pltpu
from jax.experimental.pallas import tpu_sc as plsc

_VOCAB = 1000000
_EMBED_DIM = 32
_BATCH = 4096
_HIST = 200

_NW = 32
_NTILE_FULL = _VOCAB // 128
_TAIL0 = _NTILE_FULL * 128
_NTAIL = _VOCAB - _TAIL0
_RM_ROWS = _VOCAB // 4
_KMAX = (_NTILE_FULL + _NW - 1) // _NW

_NHG = _HIST // 8
_NBT = _BATCH // 128
_NGRP = _NHG * _NBT
_GPW = _NGRP // _NW


def _detile_body(tT_hbm, tail_hbm, rm_hbm, in_v, out_v, sems_i, sems_o,
                 tail_v):
    wid = lax.axis_index("s") * 2 + lax.axis_index("c")

    iota = lax.iota(jnp.int32, 16)
    e_idx = (iota, iota + 16)

    @pl.when(wid == _NW - 1)
    def _():
        pltpu.sync_copy(tail_hbm, tail_v)
        pltpu.sync_copy(
            tail_v, rm_hbm.at[pl.ds(_RM_ROWS - _NTAIL // 4, _NTAIL // 4), :])

    def col(k):
        return k * _NW + wid

    def issue_in(k, b):
        pltpu.async_copy(
            tT_hbm.at[:, pl.ds(col(k) * 128, 128)],
            in_v[b].at[:, pl.ds(0, 128)], sems_i[b])

    def wait_in(b):
        pltpu.make_async_copy(
            tT_hbm.at[:, pl.ds(0, 128)],
            in_v[b].at[:, pl.ds(0, 128)], sems_i[b]).wait()

    def issue_out(k, b):
        pltpu.async_copy(
            out_v[b], rm_hbm.at[pl.ds(col(k) * 32, 32), :], sems_o[b])

    def wait_out(b):
        pltpu.make_async_copy(
            out_v[b], rm_hbm.at[pl.ds(0, 32), :], sems_o[b]).wait()

    def transpose(b):
        for r in range(32):
            for p in range(8):
                c_idx = jnp.full((16,), 4 * r + p // 2, jnp.int32)
                out_v[b][r, pl.ds(16 * p, 16)] = plsc.load_gather(
                    in_v[b], [e_idx[p % 2], c_idx])

    issue_in(0, 0)

    def step(k2, carry):
        for par in range(2):
            k = k2 * 2 + par

            @pl.when(col(k) < _NTILE_FULL)
            def _():
                wait_in(par)

                @pl.when(col(k + 1) < _NTILE_FULL)
                def _():
                    issue_in(k + 1, 1 - par)

                @pl.when(k >= 2)
                def _():
                    wait_out(par)
                transpose(par)
                issue_out(k, par)
        return carry

    lax.fori_loop(0, _KMAX // 2, step, 0)
    k_last = _KMAX - 1
    p_last = k_last % 2

    @pl.when(col(k_last) < _NTILE_FULL)
    def _():
        wait_in(p_last)
        wait_out(p_last)
        transpose(p_last)
        issue_out(k_last, p_last)

    wait_out(0)
    wait_out(1)


def _gather_body(rm_hbm, x3_hbm, out_hbm, idx_v, rows_v, ob_v,
                 sem_i, sems_g, sems_o):
    wid = lax.axis_index("s") * 2 + lax.axis_index("c")

    iota = lax.iota(jnp.int32, 16)
    r_idx = tuple(iota + 16 * p for p in range(8))

    def issue_idx(g):
        grp = wid * _GPW + g
        pltpu.async_copy(
            x3_hbm.at[pl.ds((grp // _NBT) * 8, 8), lax.rem(grp, _NBT), :],
            idx_v.at[lax.rem(g, 2)], sem_i)

    def wait_idx():
        pltpu.make_async_copy(
            x3_hbm.at[pl.ds(0, 8), 0, :], idx_v.at[0], sem_i).wait()

    def issue_gather(g, s, par):
        pltpu.async_copy(
            rm_hbm.at[idx_v.at[lax.rem(g, 2), s]], rows_v[par], sems_g[par])

    def wait_gather(par):
        pltpu.make_async_copy(
            rm_hbm.at[idx_v.at[0, 0]], rows_v[par], sems_g[par]).wait()

    def issue_out(g, s, par):
        grp = wid * _GPW + g
        pltpu.async_copy(
            ob_v[par].at[:, :, pl.ds(0, 128)],
            out_hbm.at[(grp // _NBT) * 8 + s, :, lax.rem(grp, _NBT)],
            sems_o[par])

    def wait_out(par):
        pltpu.make_async_copy(
            ob_v[par].at[:, :, pl.ds(0, 128)],
            out_hbm.at[0, :, 0], sems_o[par]).wait()

    r3_idx = tuple((iota + 16 * p2) >> 3 for p2 in range(2))
    s3_idx = tuple((iota + 16 * p2) & 7 for p2 in range(2))

    def transpose(par):
        for l in range(128):
            for p2 in range(2):
                vals = rows_v[par][l, pl.ds(16 * p2, 16)]
                plsc.store_scatter(
                    ob_v[par],
                    [r3_idx[p2], s3_idx[p2], jnp.full((16,), l, jnp.int32)],
                    vals)

    grp0 = wid * _GPW
    pltpu.sync_copy(
        x3_hbm.at[pl.ds((grp0 // _NBT) * 8, 8), lax.rem(grp0, _NBT), :],
        idx_v.at[0])
    issue_gather(0, 0, 0)
    issue_gather(0, 1, 1)

    def gstep(g, carry):
        @pl.when(g + 1 < _GPW)
        def _():
            issue_idx(g + 1)
        for s in range(8):
            par = s % 2
            wait_gather(par)

            @pl.when(8 * g + s >= 2)
            def _():
                wait_out(par)
            transpose(par)
            issue_out(g, s, par)
            if s == 6:
                @pl.when(g + 1 < _GPW)
                def _():
                    wait_idx()
            if s < 6:
                @pl.when(8 * g + s + 2 < 8 * _GPW)
                def _():
                    issue_gather(g, s + 2, par)
            else:
                @pl.when(g + 1 < _GPW)
                def _():
                    issue_gather(g + 1, s - 6, par)
        return carry

    lax.fori_loop(0, _GPW, gstep, 0)

    wait_out(0)
    wait_out(1)


@jax.jit
def _embedding(x, table):
    mesh = plsc.VectorSubcoreMesh(core_axis_name="c", subcore_axis_name="s")

    detile = pl.kernel(
        _detile_body,
        out_type=jax.ShapeDtypeStruct((_RM_ROWS, 128), jnp.float32),
        mesh=mesh,
        scratch_types=[
            [pltpu.VMEM((32, 129), jnp.float32) for _ in range(2)],
            [pltpu.VMEM((32, 128), jnp.float32) for _ in range(2)],
            [pltpu.SemaphoreType.DMA for _ in range(2)],
            [pltpu.SemaphoreType.DMA for _ in range(2)],
            pltpu.VMEM((_NTAIL // 4, 128), jnp.float32),
        ],
        compiler_params=pltpu.CompilerParams(
            use_tc_tiling_on_sc=True, needs_layout_passes=False),
    )

    gather = pl.kernel(
        _gather_body,
        out_type=jax.ShapeDtypeStruct((_HIST, 4, _NBT, 8, 128), jnp.float32),
        mesh=mesh,
        scratch_types=[
            pltpu.VMEM((2, 8, 128), jnp.int32),
            [pltpu.VMEM((128, _EMBED_DIM), jnp.float32) for _ in range(2)],
            [pltpu.VMEM((4, 8, 133), jnp.float32) for _ in range(2)],
            pltpu.SemaphoreType.DMA,
            [pltpu.SemaphoreType.DMA for _ in range(2)],
            [pltpu.SemaphoreType.DMA for _ in range(2)],
        ],
        compiler_params=pltpu.CompilerParams(
            use_tc_tiling_on_sc=False, needs_layout_passes=False),
    )

    tT = table.T
    tail = table[_TAIL0:, :].reshape(_NTAIL // 4, 128)
    rm = detile(tT, tail)
    rm_lin = rm.reshape(_VOCAB, _EMBED_DIM)
    x3 = x.T.reshape(_HIST, _NBT, 128)
    out_lin = gather(rm_lin, x3)
    return out_lin.transpose(2, 4, 0, 1, 3).reshape(_BATCH, _HIST, _EMBED_DIM)


def kernel(x, table):
    return _embedding(x, table)


# speedup vs baseline: 2.5260x; 1.4700x over previous
import jax
import jax.numpy as jnp
from jax import lax
from jax.experimental import pallas as pl
from jax.experimental.pallas import tpu as pltpu
from jax.experimental.pallas import tpu_sc as plsc

_VOCAB = 1000000
_EMBED_DIM = 32
_BATCH = 4096
_HIST = 200

_NW = 32
_NTILE_FULL = _VOCAB // 128
_TAIL0 = _NTILE_FULL * 128
_NTAIL = _VOCAB - _TAIL0
_RM_ROWS = _VOCAB // 4
_KMAX = (_NTILE_FULL + _NW - 1) // _NW

_NHG = _HIST // 8
_NBT = _BATCH // 128
_NGRP = _NHG * _NBT
_GPW = _NGRP // _NW


def _detile_body(tT_hbm, tail_hbm, rm_hbm, in_v, out_v, sems_i, sems_o,
                 tail_v):
    wid = lax.axis_index("s") * 2 + lax.axis_index("c")

    iota = lax.iota(jnp.int32, 16)
    e_idx = (iota, iota + 16)

    @pl.when(wid == _NW - 1)
    def _():
        pltpu.sync_copy(tail_hbm, tail_v)
        pltpu.sync_copy(
            tail_v, rm_hbm.at[pl.ds(_RM_ROWS - _NTAIL // 4, _NTAIL // 4), :])

    def col(k):
        return k * _NW + wid

    def issue_in(k, b):
        pltpu.async_copy(
            tT_hbm.at[:, pl.ds(col(k) * 128, 128)],
            in_v[b].at[:, pl.ds(0, 128)], sems_i[b])

    def wait_in(b):
        pltpu.make_async_copy(
            tT_hbm.at[:, pl.ds(0, 128)],
            in_v[b].at[:, pl.ds(0, 128)], sems_i[b]).wait()

    def issue_out(k, b):
        pltpu.async_copy(
            out_v[b], rm_hbm.at[pl.ds(col(k) * 32, 32), :], sems_o[b])

    def wait_out(b):
        pltpu.make_async_copy(
            out_v[b], rm_hbm.at[pl.ds(0, 32), :], sems_o[b]).wait()

    def transpose(b):
        for r in range(32):
            for p in range(8):
                c_idx = jnp.full((16,), 4 * r + p // 2, jnp.int32)
                out_v[b][r, pl.ds(16 * p, 16)] = plsc.load_gather(
                    in_v[b], [e_idx[p % 2], c_idx])

    issue_in(0, 0)

    def step(k2, carry):
        for par in range(2):
            k = k2 * 2 + par

            @pl.when(col(k) < _NTILE_FULL)
            def _():
                wait_in(par)

                @pl.when(col(k + 1) < _NTILE_FULL)
                def _():
                    issue_in(k + 1, 1 - par)

                @pl.when(k >= 2)
                def _():
                    wait_out(par)
                transpose(par)
                issue_out(k, par)
        return carry

    lax.fori_loop(0, _KMAX // 2, step, 0)
    k_last = _KMAX - 1
    p_last = k_last % 2

    @pl.when(col(k_last) < _NTILE_FULL)
    def _():
        wait_in(p_last)
        wait_out(p_last)
        transpose(p_last)
        issue_out(k_last, p_last)

    wait_out(0)
    wait_out(1)


def _gather_body(rm_hbm, x3_hbm, out_hbm, idx_v, rows_v, ob_v,
                 sem_i, sems_g, sems_o):
    wid = lax.axis_index("s") * 2 + lax.axis_index("c")

    iota = lax.iota(jnp.int32, 16)
    r_idx = tuple(iota + 16 * p for p in range(8))

    def issue_idx(g):
        grp = wid * _GPW + g
        pltpu.async_copy(
            x3_hbm.at[pl.ds((grp // _NBT) * 8, 8), lax.rem(grp, _NBT), :],
            idx_v.at[lax.rem(g, 2)], sem_i)

    def wait_idx():
        pltpu.make_async_copy(
            x3_hbm.at[pl.ds(0, 8), 0, :], idx_v.at[0], sem_i).wait()

    def issue_gather(g, s, par):
        pltpu.async_copy(
            rm_hbm.at[idx_v.at[lax.rem(g, 2), s]], rows_v[par], sems_g[par])

    def wait_gather(par):
        pltpu.make_async_copy(
            rm_hbm.at[idx_v.at[0, 0]], rows_v[par], sems_g[par]).wait()

    def issue_out(g, s, par):
        grp = wid * _GPW + g
        pltpu.async_copy(
            ob_v[par].at[:, :, pl.ds(0, 128)],
            out_hbm.at[(grp // _NBT) * 8 + s, :, lax.rem(grp, _NBT)],
            sems_o[par])

    def wait_out(par):
        pltpu.make_async_copy(
            ob_v[par].at[:, :, pl.ds(0, 128)],
            out_hbm.at[0, :, 0], sems_o[par]).wait()

    r3_idx = tuple((iota + 16 * p2) >> 3 for p2 in range(2))
    s3_idx = tuple((iota + 16 * p2) & 7 for p2 in range(2))

    def transpose(par):
        for l in range(128):
            for p2 in range(2):
                vals = rows_v[par][l, pl.ds(16 * p2, 16)]
                plsc.store_scatter(
                    ob_v[par],
                    [r3_idx[p2], s3_idx[p2], jnp.full((16,), l, jnp.int32)],
                    vals)

    grp0 = wid * _GPW
    pltpu.sync_copy(
        x3_hbm.at[pl.ds((grp0 // _NBT) * 8, 8), lax.rem(grp0, _NBT), :],
        idx_v.at[0])
    issue_gather(0, 0, 0)
    issue_gather(0, 1, 1)

    def gstep(g, carry):
        @pl.when(g + 1 < _GPW)
        def _():
            issue_idx(g + 1)
        for s in range(8):
            par = s % 2
            wait_gather(par)

            @pl.when(8 * g + s >= 2)
            def _():
                wait_out(par)
            transpose(par)
            issue_out(g, s, par)
            if s == 6:
                @pl.when(g + 1 < _GPW)
                def _():
                    wait_idx()
            if s < 6:
                @pl.when(8 * g + s + 2 < 8 * _GPW)
                def _():
                    issue_gather(g, s + 2, par)
            else:
                @pl.when(g + 1 < _GPW)
                def _():
                    issue_gather(g + 1, s - 6, par)
        return carry

    lax.fori_loop(0, _GPW, gstep, 0)

    wait_out(0)
    wait_out(1)


@jax.jit
def _embedding(x, table):
    mesh = plsc.VectorSubcoreMesh(core_axis_name="c", subcore_axis_name="s")

    detile = pl.kernel(
        _detile_body,
        out_type=jax.ShapeDtypeStruct((_RM_ROWS, 128), jnp.float32),
        mesh=mesh,
        scratch_types=[
            [pltpu.VMEM((32, 129), jnp.float32) for _ in range(2)],
            [pltpu.VMEM((32, 128), jnp.float32) for _ in range(2)],
            [pltpu.SemaphoreType.DMA for _ in range(2)],
            [pltpu.SemaphoreType.DMA for _ in range(2)],
            pltpu.VMEM((_NTAIL // 4, 128), jnp.float32),
        ],
        compiler_params=pltpu.CompilerParams(
            use_tc_tiling_on_sc=True, needs_layout_passes=False),
    )

    gather = pl.kernel(
        _gather_body,
        out_type=jax.ShapeDtypeStruct((_HIST, 4, _NBT, 8, 128), jnp.float32),
        mesh=mesh,
        scratch_types=[
            pltpu.VMEM((2, 8, 128), jnp.int32),
            [pltpu.VMEM((128, _EMBED_DIM), jnp.float32) for _ in range(2)],
            [pltpu.VMEM((4, 8, 133), jnp.float32) for _ in range(2)],
            pltpu.SemaphoreType.DMA,
            [pltpu.SemaphoreType.DMA for _ in range(2)],
            [pltpu.SemaphoreType.DMA for _ in range(2)],
        ],
        compiler_params=pltpu.CompilerParams(
            use_tc_tiling_on_sc=False, needs_layout_passes=False),
    )

    rm_lin = table
    x3 = x.T.reshape(_HIST, _NBT, 128)
    out_lin = gather(rm_lin, x3)
    return out_lin.transpose(2, 4, 0, 1, 3).reshape(_BATCH, _HIST, _EMBED_DIM)


def kernel(x, table):
    return _embedding(x, table)


# speedup vs baseline: 3.5481x; 1.4046x over previous
import jax
import jax.numpy as jnp
from jax import lax
from jax.experimental import pallas as pl
from jax.experimental.pallas import tpu as pltpu
from jax.experimental.pallas import tpu_sc as plsc

_VOCAB = 1000000
_EMBED_DIM = 32
_BATCH = 4096
_HIST = 200

_NW = 32
_NTILE_FULL = _VOCAB // 128
_TAIL0 = _NTILE_FULL * 128
_NTAIL = _VOCAB - _TAIL0
_RM_ROWS = _VOCAB // 4
_KMAX = (_NTILE_FULL + _NW - 1) // _NW

_NHG = _HIST // 8
_NBT = _BATCH // 128
_NGRP = _NHG * _NBT
_GPW = _NGRP // _NW


def _detile_body(tT_hbm, tail_hbm, rm_hbm, in_v, out_v, sems_i, sems_o,
                 tail_v):
    wid = lax.axis_index("s") * 2 + lax.axis_index("c")

    iota = lax.iota(jnp.int32, 16)
    e_idx = (iota, iota + 16)

    @pl.when(wid == _NW - 1)
    def _():
        pltpu.sync_copy(tail_hbm, tail_v)
        pltpu.sync_copy(
            tail_v, rm_hbm.at[pl.ds(_RM_ROWS - _NTAIL // 4, _NTAIL // 4), :])

    def col(k):
        return k * _NW + wid

    def issue_in(k, b):
        pltpu.async_copy(
            tT_hbm.at[:, pl.ds(col(k) * 128, 128)], in_v[b], sems_i[b])

    def wait_in(b):
        pltpu.make_async_copy(
            tT_hbm.at[:, pl.ds(0, 128)], in_v[b], sems_i[b]).wait()

    def issue_out(k, b):
        pltpu.async_copy(
            out_v[b], rm_hbm.at[pl.ds(col(k) * 32, 32), :], sems_o[b])

    def wait_out(b):
        pltpu.make_async_copy(
            out_v[b], rm_hbm.at[pl.ds(0, 32), :], sems_o[b]).wait()

    def transpose(b):
        def diag(d, carry):
            perm = (iota + d) & 15
            pr2 = perm >> 2
            pl_ = ((perm & 3) << 5) + iota
            for e0 in (0, 16):
                for c0 in range(0, 128, 16):
                    vals = plsc.load_gather(
                        in_v[b], [e_idx[e0 // 16], perm + c0])
                    plsc.store_scatter(
                        out_v[b], [pr2 + (c0 >> 2), pl_ + e0], vals)
            return carry

        lax.fori_loop(0, 16, diag, 0)

    issue_in(0, 0)

    def step(k2, carry):
        for par in range(2):
            k = k2 * 2 + par

            @pl.when(col(k) < _NTILE_FULL)
            def _():
                wait_in(par)

                @pl.when(col(k + 1) < _NTILE_FULL)
                def _():
                    issue_in(k + 1, 1 - par)

                @pl.when(k >= 2)
                def _():
                    wait_out(par)
                transpose(par)
                issue_out(k, par)
        return carry

    lax.fori_loop(0, _KMAX // 2, step, 0)
    k_last = _KMAX - 1
    p_last = k_last % 2

    @pl.when(col(k_last) < _NTILE_FULL)
    def _():
        wait_in(p_last)
        wait_out(p_last)
        transpose(p_last)
        issue_out(k_last, p_last)

    wait_out(0)
    wait_out(1)


def _gather_body(rm_hbm, x3_hbm, out_hbm, idx_v, rows_v, ob_v,
                 sem_i, sems_g, sems_o):
    wid = lax.axis_index("s") * 2 + lax.axis_index("c")

    iota = lax.iota(jnp.int32, 16)
    r_idx = tuple(iota + 16 * p for p in range(8))

    def issue_idx(g):
        grp = wid * _GPW + g
        pltpu.async_copy(
            x3_hbm.at[pl.ds((grp // _NBT) * 8, 8), lax.rem(grp, _NBT), :],
            idx_v.at[lax.rem(g, 2)], sem_i)

    def wait_idx():
        pltpu.make_async_copy(
            x3_hbm.at[pl.ds(0, 8), 0, :], idx_v.at[0], sem_i).wait()

    def issue_gather(g, s, par):
        pltpu.async_copy(
            rm_hbm.at[idx_v.at[lax.rem(g, 2), s]], rows_v[par], sems_g[par])

    def wait_gather(par):
        pltpu.make_async_copy(
            rm_hbm.at[idx_v.at[0, 0]], rows_v[par], sems_g[par]).wait()

    def issue_out(g, s, par):
        grp = wid * _GPW + g
        pltpu.async_copy(
            ob_v[par].at[:, :, pl.ds(0, 128)],
            out_hbm.at[(grp // _NBT) * 8 + s, :, lax.rem(grp, _NBT)],
            sems_o[par])

    def wait_out(par):
        pltpu.make_async_copy(
            ob_v[par].at[:, :, pl.ds(0, 128)],
            out_hbm.at[0, :, 0], sems_o[par]).wait()

    r3_idx = tuple((iota + 16 * p2) >> 3 for p2 in range(2))
    s3_idx = tuple((iota + 16 * p2) & 7 for p2 in range(2))

    def transpose(par):
        for l in range(128):
            for p2 in range(2):
                vals = rows_v[par][l, pl.ds(16 * p2, 16)]
                plsc.store_scatter(
                    ob_v[par],
                    [r3_idx[p2], s3_idx[p2], jnp.full((16,), l, jnp.int32)],
                    vals)

    grp0 = wid * _GPW
    pltpu.sync_copy(
        x3_hbm.at[pl.ds((grp0 // _NBT) * 8, 8), lax.rem(grp0, _NBT), :],
        idx_v.at[0])
    issue_gather(0, 0, 0)
    issue_gather(0, 1, 1)

    def gstep(g, carry):
        @pl.when(g + 1 < _GPW)
        def _():
            issue_idx(g + 1)
        for s in range(8):
            par = s % 2
            wait_gather(par)

            @pl.when(8 * g + s >= 2)
            def _():
                wait_out(par)
            transpose(par)
            issue_out(g, s, par)
            if s == 6:
                @pl.when(g + 1 < _GPW)
                def _():
                    wait_idx()
            if s < 6:
                @pl.when(8 * g + s + 2 < 8 * _GPW)
                def _():
                    issue_gather(g, s + 2, par)
            else:
                @pl.when(g + 1 < _GPW)
                def _():
                    issue_gather(g + 1, s - 6, par)
        return carry

    lax.fori_loop(0, _GPW, gstep, 0)

    wait_out(0)
    wait_out(1)


@jax.jit
def _embedding(x, table):
    mesh = plsc.VectorSubcoreMesh(core_axis_name="c", subcore_axis_name="s")

    detile = pl.kernel(
        _detile_body,
        out_type=jax.ShapeDtypeStruct((_RM_ROWS, 128), jnp.float32),
        mesh=mesh,
        scratch_types=[
            [pltpu.VMEM((32, 128), jnp.float32) for _ in range(2)],
            [pltpu.VMEM((32, 128), jnp.float32) for _ in range(2)],
            [pltpu.SemaphoreType.DMA for _ in range(2)],
            [pltpu.SemaphoreType.DMA for _ in range(2)],
            pltpu.VMEM((_NTAIL // 4, 128), jnp.float32),
        ],
        compiler_params=pltpu.CompilerParams(
            use_tc_tiling_on_sc=True, needs_layout_passes=False),
    )

    gather = pl.kernel(
        _gather_body,
        out_type=jax.ShapeDtypeStruct((_HIST, 4, _NBT, 8, 128), jnp.float32),
        mesh=mesh,
        scratch_types=[
            pltpu.VMEM((2, 8, 128), jnp.int32),
            [pltpu.VMEM((128, _EMBED_DIM), jnp.float32) for _ in range(2)],
            [pltpu.VMEM((4, 8, 133), jnp.float32) for _ in range(2)],
            pltpu.SemaphoreType.DMA,
            [pltpu.SemaphoreType.DMA for _ in range(2)],
            [pltpu.SemaphoreType.DMA for _ in range(2)],
        ],
        compiler_params=pltpu.CompilerParams(
            use_tc_tiling_on_sc=False, needs_layout_passes=False),
    )

    tT = table.T
    tail = table[_TAIL0:, :].reshape(_NTAIL // 4, 128)
    rm = detile(tT, tail)
    rm_lin = rm.reshape(_VOCAB, _EMBED_DIM)
    x3 = x.T.reshape(_HIST, _NBT, 128)
    out_lin = gather(rm_lin, x3)
    return out_lin.transpose(2, 4, 0, 1, 3).reshape(_BATCH, _HIST, _EMBED_DIM)


def kernel(x, table):
    return _embedding(x, table)
